# baseline (speedup 1.0000x reference)
import functools

import jax
import jax.numpy as jnp
from jax import lax
from jax.experimental import pallas as pl
from jax.experimental.pallas import tpu as pltpu
from jax.experimental.pallas import tpu_sc as plsc

VOCAB = 1000000
DIM = 64
BATCH = 16384
HIST = 20

NC, NS = 2, 16
NW = NC * NS

GRP = 128
NBLK = BATCH // GRP
K_PER_W = NBLK // NW
U_PER_W = HIST * K_PER_W
NBUF = 4


def _build():
    mesh = plsc.VectorSubcoreMesh(core_axis_name="c", subcore_axis_name="s")

    @functools.partial(
        pl.kernel,
        mesh=mesh,
        out_type=jax.ShapeDtypeStruct((HIST, 8, NBLK, 8, GRP), jnp.float32),
        scratch_types=[
            pltpu.VMEM((U_PER_W, GRP), jnp.int32),
            pltpu.VMEM((NBUF, GRP, DIM), jnp.float32),
            pltpu.VMEM((2, 8, K_PER_W, 8, GRP), jnp.float32),
            pltpu.SemaphoreType.DMA,
            pltpu.SemaphoreType.DMA,
            pltpu.SemaphoreType.DMA,
            pltpu.SemaphoreType.DMA,
            pltpu.SemaphoreType.DMA,
            pltpu.SemaphoreType.DMA,
            pltpu.SemaphoreType.DMA,
        ],
        compiler_params=pltpu.CompilerParams(
            use_tc_tiling_on_sc=False, needs_layout_passes=False
        ),
    )
    def gather_kernel(idx_hbm, table_hbm, out_hbm, idx_v, rows_v, trans_v,
                      isem, g0, g1, g2, g3, w0, w1):
        wid = lax.axis_index("s") * NC + lax.axis_index("c")
        gsem = (g0, g1, g2, g3)
        wsem = (w0, w1)
        lanes = lax.iota(jnp.int32, 16)

        def unit_hk(u):
            return lax.shift_right_logical(u, 2), lax.bitwise_and(u, 3)

        def idx_load(u, carry):
            h, k = unit_hk(u)
            pltpu.async_copy(
                idx_hbm.at[h, pl.ds((wid * K_PER_W + k) * GRP, GRP)],
                idx_v.at[u], isem,
            )
            return carry

        def idx_drain(u, carry):
            pltpu.make_async_copy(
                idx_hbm.at[0, pl.ds(0, GRP)], idx_v.at[0], isem
            ).wait()
            return carry

        lax.fori_loop(0, U_PER_W, idx_load, 0)
        lax.fori_loop(0, U_PER_W, idx_drain, 0)

        def gather_start(u, buf):
            pltpu.async_copy(table_hbm.at[idx_v.at[u]], rows_v.at[buf], gsem[buf])

        def gather_wait(buf):
            pltpu.make_async_copy(
                table_hbm.at[pl.ds(0, GRP)], rows_v.at[buf], gsem[buf]
            ).wait()

        def write_start(h, buf):
            pltpu.async_copy(
                trans_v.at[buf],
                out_hbm.at[h, :, pl.ds(wid * K_PER_W, K_PER_W)],
                wsem[buf],
            )

        def write_wait(buf):
            pltpu.make_async_copy(
                trans_v.at[buf], out_hbm.at[0, :, pl.ds(0, K_PER_W)], wsem[buf]
            ).wait()

        def transpose(buf, tb, k):
            rows = rows_v.at[buf]
            trans = trans_v.at[tb]

            def cb_body(cb, carry):
                for half in range(2):
                    c0 = pl.multiple_of((2 * cb + half) * 16, 16)
                    ri = c0 + lanes
                    for dg in range(0, DIM // 8, 2):
                        vals = [
                            plsc.load_gather(
                                rows, [ri, jnp.full((16,), dg * 8 + t, jnp.int32)]
                            )
                            for t in range(16)
                        ]
                        for t in range(16):
                            trans[dg + t // 8, k, t % 8, pl.ds(c0, 16)] = vals[t]
                return carry

            lax.fori_loop(0, GRP // 32, cb_body, 0)

        for j in range(NBUF):
            gather_start(j, j)

        def body(i, carry):
            for tb in range(2):
                h = 2 * i + tb

                @pl.when(i > 0)
                def _w():
                    write_wait(tb)

                for j in range(NBUF):
                    u = NBUF * h + j
                    gather_wait(j)
                    transpose(j, tb, j)

                    @pl.when(2 * i < HIST - 1 - tb)
                    def _g():
                        gather_start(u + NBUF, j)

                write_start(h, tb)
            return carry

        lax.fori_loop(0, HIST // 2, body, 0)
        write_wait(0)
        write_wait(1)

    return gather_kernel


_gather = _build()


def kernel(indices, weight):
    wpad = jnp.pad(weight, ((0, 0), (0, 128 - DIM)))
    table = wpad.reshape(2 * VOCAB, DIM)
    idx2 = indices.T.astype(jnp.int32) * 2
    t = _gather(idx2, table)
    return t.transpose(2, 4, 0, 1, 3).reshape(BATCH, HIST, DIM)


# speedup vs baseline: 1.0187x; 1.0187x over previous
import functools

import jax
import jax.numpy as jnp
from jax import lax
from jax.experimental import pallas as pl
from jax.experimental.pallas import tpu as pltpu
from jax.experimental.pallas import tpu_sc as plsc

VOCAB = 1000000
DIM = 64
BATCH = 16384
HIST = 20

NC, NS = 2, 16
NW = NC * NS

GRP = 128
NBLK = BATCH // GRP
K_PER_W = NBLK // NW
U_PER_W = HIST * K_PER_W
NBUF = 4


def _build():
    mesh = plsc.VectorSubcoreMesh(core_axis_name="c", subcore_axis_name="s")

    @functools.partial(
        pl.kernel,
        mesh=mesh,
        out_type=jax.ShapeDtypeStruct((HIST, 8, NBLK, 8, GRP), jnp.float32),
        scratch_types=[
            pltpu.VMEM((U_PER_W, GRP), jnp.int32),
            pltpu.VMEM((NBUF, GRP, DIM), jnp.float32),
            pltpu.VMEM((2, K_PER_W, 8, 8, GRP), jnp.float32),
            pltpu.SemaphoreType.DMA,
            pltpu.SemaphoreType.DMA,
            pltpu.SemaphoreType.DMA,
            pltpu.SemaphoreType.DMA,
            pltpu.SemaphoreType.DMA,
            pltpu.SemaphoreType.DMA,
            pltpu.SemaphoreType.DMA,
        ],
        compiler_params=pltpu.CompilerParams(
            use_tc_tiling_on_sc=False, needs_layout_passes=False
        ),
    )
    def gather_kernel(idx_hbm, table_hbm, out_hbm, idx_v, rows_v, trans_v,
                      isem, g0, g1, g2, g3, w0, w1):
        wid = lax.axis_index("s") * NC + lax.axis_index("c")
        gsem = (g0, g1, g2, g3)
        wsem = (w0, w1)
        lanes = lax.iota(jnp.int32, 16)

        def unit_hk(u):
            return lax.shift_right_logical(u, 2), lax.bitwise_and(u, 3)

        def idx_load(u, carry):
            h, k = unit_hk(u)
            pltpu.async_copy(
                idx_hbm.at[h, pl.ds((wid * K_PER_W + k) * GRP, GRP)],
                idx_v.at[u], isem,
            )
            return carry

        def idx_drain(u, carry):
            pltpu.make_async_copy(
                idx_hbm.at[0, pl.ds(0, GRP)], idx_v.at[0], isem
            ).wait()
            return carry

        lax.fori_loop(0, U_PER_W, idx_load, 0)
        lax.fori_loop(0, U_PER_W, idx_drain, 0)

        def gather_start(u, buf):
            pltpu.async_copy(table_hbm.at[idx_v.at[u]], rows_v.at[buf], gsem[buf])

        def gather_wait(buf):
            pltpu.make_async_copy(
                table_hbm.at[pl.ds(0, GRP)], rows_v.at[buf], gsem[buf]
            ).wait()

        def write_start(h, buf):
            for k in range(K_PER_W):
                pltpu.async_copy(
                    trans_v.at[buf, k],
                    out_hbm.at[h, :, wid * K_PER_W + k],
                    wsem[buf],
                )

        def write_wait(buf):
            for k in range(K_PER_W):
                pltpu.make_async_copy(
                    trans_v.at[buf, 0], out_hbm.at[0, :, 0], wsem[buf]
                ).wait()

        def transpose(buf, tb, k):
            rows = rows_v.at[buf]
            trans = trans_v.at[tb, k]

            def cb_body(cb, carry):
                c0 = pl.multiple_of(cb * 16, 16)
                ri = c0 + lanes
                for j in range(16):
                    m = lax.bitwise_and(lanes + j, 15)
                    mhi = lax.shift_right_logical(m, 3)
                    mlo = lax.bitwise_and(m, 7)
                    for mb in range(4):
                        ci = 16 * mb + m
                        rgrp = 2 * mb + mhi
                        vals = plsc.load_gather(rows, [ri, ci])
                        plsc.store_scatter(trans, [rgrp, mlo, ri], vals)
                return carry

            lax.fori_loop(0, GRP // 16, cb_body, 0)

        for j in range(NBUF):
            gather_start(j, j)

        def body(i, carry):
            for tb in range(2):
                h = 2 * i + tb

                @pl.when(i > 0)
                def _w():
                    write_wait(tb)

                for j in range(NBUF):
                    u = NBUF * h + j
                    gather_wait(j)
                    transpose(j, tb, j)

                    @pl.when(2 * i < HIST - 1 - tb)
                    def _g():
                        gather_start(u + NBUF, j)

                write_start(h, tb)
            return carry

        lax.fori_loop(0, HIST // 2, body, 0)
        write_wait(0)
        write_wait(1)

    return gather_kernel


_gather = _build()


def kernel(indices, weight):
    wpad = jnp.pad(weight, ((0, 0), (0, 128 - DIM)))
    table = wpad.reshape(2 * VOCAB, DIM)
    idx2 = indices.T.astype(jnp.int32) * 2
    t = _gather(idx2, table)
    return t.transpose(2, 4, 0, 1, 3).reshape(BATCH, HIST, DIM)


# speedup vs baseline: 1.2238x; 1.2013x over previous
import functools

import jax
import jax.numpy as jnp
from jax import lax
from jax.experimental import pallas as pl
from jax.experimental.pallas import tpu as pltpu
from jax.experimental.pallas import tpu_sc as plsc

VOCAB = 1000000
DIM = 64
BATCH = 16384
HIST = 20

NC, NS = 2, 16
NW = NC * NS

GRP = 128
NBLK = BATCH // GRP
K_PER_W = NBLK // NW
U_PER_W = HIST * K_PER_W
NBUF = 4


def _build():
    mesh = plsc.VectorSubcoreMesh(core_axis_name="c", subcore_axis_name="s")

    @functools.partial(
        pl.kernel,
        mesh=mesh,
        out_type=jax.ShapeDtypeStruct((HIST, 8, NBLK, 8, GRP), jnp.float32),
        scratch_types=[
            pltpu.VMEM((U_PER_W, GRP), jnp.int32),
            pltpu.VMEM((NBUF, GRP, DIM), jnp.float32),
            pltpu.VMEM((2, K_PER_W, 8, 8, GRP), jnp.float32),
            pltpu.SemaphoreType.DMA,
            pltpu.SemaphoreType.DMA,
            pltpu.SemaphoreType.DMA,
            pltpu.SemaphoreType.DMA,
            pltpu.SemaphoreType.DMA,
            pltpu.SemaphoreType.DMA,
            pltpu.SemaphoreType.DMA,
        ],
        compiler_params=pltpu.CompilerParams(
            use_tc_tiling_on_sc=False, needs_layout_passes=False
        ),
    )
    def gather_kernel(idx_hbm, table_hbm, out_hbm, idx_v, rows_v, trans_v,
                      isem, g0, g1, g2, g3, w0, w1):
        wid = lax.axis_index("s") * NC + lax.axis_index("c")
        gsem = (g0, g1, g2, g3)
        wsem = (w0, w1)
        lanes = lax.iota(jnp.int32, 16)

        def unit_hk(u):
            return lax.shift_right_logical(u, 2), lax.bitwise_and(u, 3)

        def idx_load(u, carry):
            h, k = unit_hk(u)
            pltpu.async_copy(
                idx_hbm.at[h, pl.ds((wid * K_PER_W + k) * GRP, GRP)],
                idx_v.at[u], isem,
            )
            return carry

        def idx_drain(u, carry):
            pltpu.make_async_copy(
                idx_hbm.at[0, pl.ds(0, GRP)], idx_v.at[0], isem
            ).wait()
            return carry

        lax.fori_loop(0, U_PER_W, idx_load, 0)
        lax.fori_loop(0, U_PER_W, idx_drain, 0)

        def gather_start(u, buf):
            pltpu.async_copy(table_hbm.at[idx_v.at[u]], rows_v.at[buf], gsem[buf])

        def gather_wait(buf):
            pltpu.make_async_copy(
                table_hbm.at[pl.ds(0, GRP)], rows_v.at[buf], gsem[buf]
            ).wait()

        def write_start(h, buf):
            for k in range(K_PER_W):
                pltpu.async_copy(
                    trans_v.at[buf, k],
                    out_hbm.at[h, :, wid * K_PER_W + k],
                    wsem[buf],
                )

        def write_wait(buf):
            for k in range(K_PER_W):
                pltpu.make_async_copy(
                    trans_v.at[buf, 0], out_hbm.at[0, :, 0], wsem[buf]
                ).wait()

        def transpose(buf, tb, k):
            rows = rows_v.at[buf]
            trans = trans_v.at[tb, k]

            def cb_body(cb, carry):
                c0 = pl.multiple_of(cb * 16, 16)
                ri = c0 + lanes
                for j in range(16):
                    m = lax.bitwise_and(lanes + j, 15)
                    mhi = lax.shift_right_logical(m, 3)
                    mlo = lax.bitwise_and(m, 7)
                    vals = [
                        plsc.load_gather(rows, [ri, 16 * mb + m]) for mb in range(4)
                    ]
                    for mb in range(4):
                        plsc.store_scatter(trans, [2 * mb + mhi, mlo, ri], vals[mb])
                return carry

            lax.fori_loop(0, GRP // 16, cb_body, 0)

        for j in range(NBUF):
            gather_start(j, j)

        def body(i, carry):
            for tb in range(2):
                h = 2 * i + tb

                @pl.when(i > 0)
                def _w():
                    write_wait(tb)

                for j in range(NBUF):
                    u = NBUF * h + j
                    gather_wait(j)
                    transpose(j, tb, j)

                    @pl.when(2 * i < HIST - 1 - tb)
                    def _g():
                        gather_start(u + NBUF, j)

                write_start(h, tb)
            return carry

        lax.fori_loop(0, HIST // 2, body, 0)
        write_wait(0)
        write_wait(1)

    return gather_kernel


_gather = _build()


def kernel(indices, weight):
    wpad = jnp.pad(weight, ((0, 0), (0, 128 - DIM)))
    table = wpad.reshape(2 * VOCAB, DIM)
    idx2 = indices.T.astype(jnp.int32) * 2
    t = _gather(idx2, table)
    return t.transpose(2, 4, 0, 1, 3).reshape(BATCH, HIST, DIM)


# speedup vs baseline: 1.3200x; 1.0786x over previous
import functools

import jax
import jax.numpy as jnp
from jax import lax
from jax.experimental import pallas as pl
from jax.experimental.pallas import tpu as pltpu
from jax.experimental.pallas import tpu_sc as plsc

VOCAB = 1000000
DIM = 64
BATCH = 16384
HIST = 20

NC, NS = 2, 16
NW = NC * NS

GRP = 128
NBLK = BATCH // GRP
K_PER_W = NBLK // NW
U_PER_W = HIST * K_PER_W
NBUF = 4


def _build():
    mesh = plsc.VectorSubcoreMesh(core_axis_name="c", subcore_axis_name="s")

    @functools.partial(
        pl.kernel,
        mesh=mesh,
        out_type=jax.ShapeDtypeStruct((HIST, 8, NBLK, 8, GRP), jnp.float32),
        scratch_types=[
            pltpu.VMEM((U_PER_W, GRP), jnp.int32),
            pltpu.VMEM((NBUF, GRP, DIM), jnp.float32),
            pltpu.VMEM((2, K_PER_W, 8, 8, GRP), jnp.float32),
            pltpu.SemaphoreType.DMA,
            pltpu.SemaphoreType.DMA,
            pltpu.SemaphoreType.DMA,
            pltpu.SemaphoreType.DMA,
            pltpu.SemaphoreType.DMA,
            pltpu.SemaphoreType.DMA,
            pltpu.SemaphoreType.DMA,
        ],
        compiler_params=pltpu.CompilerParams(
            use_tc_tiling_on_sc=False, needs_layout_passes=False
        ),
    )
    def gather_kernel(idx_hbm, table_hbm, out_hbm, idx_v, rows_v, trans_v,
                      isem, g0, g1, g2, g3, w0, w1):
        wid = lax.axis_index("s") * NC + lax.axis_index("c")
        gsem = (g0, g1, g2, g3)
        wsem = (w0, w1)
        lanes = lax.iota(jnp.int32, 16)

        def unit_hk(u):
            return lax.shift_right_logical(u, 2), lax.bitwise_and(u, 3)

        def idx_load(u, carry):
            h, k = unit_hk(u)
            pltpu.async_copy(
                idx_hbm.at[h, pl.ds((wid * K_PER_W + k) * GRP, GRP)],
                idx_v.at[u], isem,
            )
            return carry

        def idx_drain(u, carry):
            pltpu.make_async_copy(
                idx_hbm.at[0, pl.ds(0, GRP)], idx_v.at[0], isem
            ).wait()
            return carry

        lax.fori_loop(0, U_PER_W, idx_load, 0)
        lax.fori_loop(0, U_PER_W, idx_drain, 0)

        def gather_start(u, buf):
            pltpu.async_copy(table_hbm.at[idx_v.at[u]], rows_v.at[buf], gsem[buf])

        def gather_wait(buf):
            pltpu.make_async_copy(
                table_hbm.at[pl.ds(0, GRP)], rows_v.at[buf], gsem[buf]
            ).wait()

        def write_start(h, buf):
            for k in range(K_PER_W):
                pltpu.async_copy(
                    trans_v.at[buf, k],
                    out_hbm.at[h, :, wid * K_PER_W + k],
                    wsem[buf],
                )

        def write_wait(buf):
            for k in range(K_PER_W):
                pltpu.make_async_copy(
                    trans_v.at[buf, 0], out_hbm.at[0, :, 0], wsem[buf]
                ).wait()

        def transpose(buf, tb, k):
            rows = rows_v.at[buf]
            trans = trans_v.at[tb, k]

            def cb_body(cb, carry):
                c0 = pl.multiple_of(cb * 16, 16)
                ri = c0 + lanes
                for j0 in range(0, 16, 2):
                    ms = [lax.bitwise_and(lanes + (j0 + t), 15) for t in range(2)]
                    mhis = [lax.shift_right_logical(m, 3) for m in ms]
                    mlos = [lax.bitwise_and(m, 7) for m in ms]
                    vals = [
                        plsc.load_gather(rows, [ri, 16 * mb + ms[t]])
                        for t in range(2)
                        for mb in range(4)
                    ]
                    for t in range(2):
                        for mb in range(4):
                            plsc.store_scatter(
                                trans, [2 * mb + mhis[t], mlos[t], ri],
                                vals[t * 4 + mb],
                            )
                return carry

            lax.fori_loop(0, GRP // 16, cb_body, 0)

        for j in range(NBUF):
            gather_start(j, j)

        def body(i, carry):
            for tb in range(2):
                h = 2 * i + tb

                @pl.when(i > 0)
                def _w():
                    write_wait(tb)

                for j in range(NBUF):
                    u = NBUF * h + j
                    gather_wait(j)
                    transpose(j, tb, j)

                    @pl.when(2 * i < HIST - 1 - tb)
                    def _g():
                        gather_start(u + NBUF, j)

                write_start(h, tb)
            return carry

        lax.fori_loop(0, HIST // 2, body, 0)
        write_wait(0)
        write_wait(1)

    return gather_kernel


_gather = _build()


def kernel(indices, weight):
    wpad = jnp.pad(weight, ((0, 0), (0, 128 - DIM)))
    table = wpad.reshape(2 * VOCAB, DIM)
    idx2 = indices.T.astype(jnp.int32) * 2
    t = _gather(idx2, table)
    return t.transpose(2, 4, 0, 1, 3).reshape(BATCH, HIST, DIM)


# speedup vs baseline: 1.5283x; 1.1578x over previous
import functools

import jax
import jax.numpy as jnp
from jax import lax
from jax.experimental import pallas as pl
from jax.experimental.pallas import tpu as pltpu
from jax.experimental.pallas import tpu_sc as plsc

VOCAB = 1000000
DIM = 64
BATCH = 16384
HIST = 20

NC, NS = 2, 16
NW = NC * NS

GRP = 128
NBLK = BATCH // GRP
K_PER_W = NBLK // NW
U_PER_W = HIST * K_PER_W
NBUF = 4


def _build():
    mesh = plsc.VectorSubcoreMesh(core_axis_name="c", subcore_axis_name="s")

    @functools.partial(
        pl.kernel,
        mesh=mesh,
        out_type=jax.ShapeDtypeStruct((HIST, 8, NBLK, 8, GRP), jnp.float32),
        scratch_types=[
            pltpu.VMEM((U_PER_W, GRP), jnp.int32),
            pltpu.VMEM((NBUF, GRP, DIM), jnp.float32),
            pltpu.VMEM((2, K_PER_W, 8, 8, GRP), jnp.float32),
            pltpu.SemaphoreType.DMA,
            pltpu.SemaphoreType.DMA,
            pltpu.SemaphoreType.DMA,
            pltpu.SemaphoreType.DMA,
            pltpu.SemaphoreType.DMA,
            pltpu.SemaphoreType.DMA,
            pltpu.SemaphoreType.DMA,
        ],
        compiler_params=pltpu.CompilerParams(
            use_tc_tiling_on_sc=False, needs_layout_passes=False
        ),
    )
    def gather_kernel(idx_hbm, table_hbm, out_hbm, idx_v, rows_v, trans_v,
                      isem, g0, g1, g2, g3, w0, w1):
        wid = lax.axis_index("s") * NC + lax.axis_index("c")
        gsem = (g0, g1, g2, g3)
        wsem = (w0, w1)
        lanes = lax.iota(jnp.int32, 16)

        def unit_hk(u):
            return lax.shift_right_logical(u, 2), lax.bitwise_and(u, 3)

        def idx_load(u, carry):
            h, k = unit_hk(u)
            pltpu.async_copy(
                idx_hbm.at[h, pl.ds((wid * K_PER_W + k) * GRP, GRP)],
                idx_v.at[u], isem,
            )
            return carry

        def idx_drain(u, carry):
            pltpu.make_async_copy(
                idx_hbm.at[0, pl.ds(0, GRP)], idx_v.at[0], isem
            ).wait()
            return carry

        lax.fori_loop(0, U_PER_W, idx_load, 0)
        lax.fori_loop(0, U_PER_W, idx_drain, 0)

        def gather_start(u, buf):
            pltpu.async_copy(table_hbm.at[idx_v.at[u]], rows_v.at[buf], gsem[buf])

        def gather_wait(buf):
            pltpu.make_async_copy(
                table_hbm.at[pl.ds(0, GRP)], rows_v.at[buf], gsem[buf]
            ).wait()

        def write_start(h, buf):
            for k in range(K_PER_W):
                pltpu.async_copy(
                    trans_v.at[buf, k],
                    out_hbm.at[h, :, wid * K_PER_W + k],
                    wsem[buf],
                )

        def write_wait(buf):
            for k in range(K_PER_W):
                pltpu.make_async_copy(
                    trans_v.at[buf, 0], out_hbm.at[0, :, 0], wsem[buf]
                ).wait()

        def transpose(buf, tb, k):
            rows = rows_v.at[buf]
            trans = trans_v.at[tb, k]

            def cb_body(cb, carry):
                c0 = pl.multiple_of(cb * 16, 16)
                ri = c0 + lanes
                for j0 in range(0, 16, 4):
                    ms = [lax.bitwise_and(lanes + (j0 + t), 15) for t in range(4)]
                    mhis = [lax.shift_right_logical(m, 3) for m in ms]
                    mlos = [lax.bitwise_and(m, 7) for m in ms]
                    vals = [
                        plsc.load_gather(rows, [ri, 16 * mb + ms[t]])
                        for t in range(4)
                        for mb in range(4)
                    ]
                    for t in range(4):
                        for mb in range(4):
                            plsc.store_scatter(
                                trans, [2 * mb + mhis[t], mlos[t], ri],
                                vals[t * 4 + mb],
                            )
                return carry

            lax.fori_loop(0, GRP // 16, cb_body, 0)

        for j in range(NBUF):
            gather_start(j, j)

        def body(i, carry):
            for tb in range(2):
                h = 2 * i + tb

                @pl.when(i > 0)
                def _w():
                    write_wait(tb)

                for j in range(NBUF):
                    u = NBUF * h + j
                    gather_wait(j)
                    transpose(j, tb, j)

                    @pl.when(2 * i < HIST - 1 - tb)
                    def _g():
                        gather_start(u + NBUF, j)

                write_start(h, tb)
            return carry

        lax.fori_loop(0, HIST // 2, body, 0)
        write_wait(0)
        write_wait(1)

    return gather_kernel


_gather = _build()


def kernel(indices, weight):
    wpad = jnp.pad(weight, ((0, 0), (0, 128 - DIM)))
    table = wpad.reshape(2 * VOCAB, DIM)
    idx2 = indices.T.astype(jnp.int32) * 2
    t = _gather(idx2, table)
    return t.transpose(2, 4, 0, 1, 3).reshape(BATCH, HIST, DIM)
